# baseline (speedup 1.0000x reference)
import functools

import jax
import jax.numpy as jnp
from jax import lax
from jax.experimental import pallas as pl
from jax.experimental.pallas import tpu as pltpu
from jax.experimental.pallas import tpu_sc as plsc

N_NODES = 10000
N_EDGES = 320000
D_IN = 128
D_OUT = 128

N_TILES = 16
N_WORKERS = 32
CHUNK = 128
CHUNKS_PER_WORKER = 80
E_PAD = N_WORKERS * CHUNKS_PER_WORKER * CHUNK
EDGES_PER_WORKER = E_PAD // N_WORKERS
N_PAD = 10240
NODES_PER_TILE = N_PAD // N_TILES

MM_BLOCK = 640


def _mm_body(f_ref, w_ref, o_ref):
    o_ref[...] = jnp.dot(f_ref[...], w_ref[...],
                         preferred_element_type=jnp.float32)


_matmul = pl.pallas_call(
    _mm_body,
    grid=(N_PAD // MM_BLOCK,),
    in_specs=[
        pl.BlockSpec((MM_BLOCK, D_IN), lambda i: (i, 0)),
        pl.BlockSpec((D_IN, D_OUT), lambda i: (0, 0)),
    ],
    out_specs=pl.BlockSpec((MM_BLOCK, D_OUT), lambda i: (i, 0)),
    out_shape=jax.ShapeDtypeStruct((N_PAD, D_OUT), jnp.float32),
)


_sc_mesh = plsc.VectorSubcoreMesh(core_axis_name="c", subcore_axis_name="s")


@functools.partial(
    pl.kernel,
    out_type=(
        jax.ShapeDtypeStruct((N_PAD, D_OUT), jnp.float32),
        jax.ShapeDtypeStruct((N_PAD, D_OUT), jnp.float32),
        jax.ShapeDtypeStruct((N_WORKERS, N_PAD), jnp.float32),
    ),
    mesh=_sc_mesh,
    compiler_params=pltpu.CompilerParams(needs_layout_passes=False),
    scratch_types=(
        [pltpu.VMEM((CHUNK,), jnp.int32)] * 4
        + [pltpu.VMEM((CHUNK,), jnp.int32)] * 4
        + [pltpu.VMEM((CHUNK, D_OUT), jnp.float32)] * 2
        + [
            pltpu.VMEM((N_PAD,), jnp.float32),
            pltpu.VMEM_SHARED((N_PAD, D_OUT), jnp.float32),
        ]
        + [pltpu.SemaphoreType.DMA] * 8
    ),
)
def _sc_scatter(h, src1d, dst1d, zeros_h,
                acc0_out, acc1_out, deg_out,
                s0, s1, s2, s3, d0, d1, d2, d3, m0, m1,
                deg_t, acc_sh,
                i0, i1, i2, i3, g0, g1, t0, t1):
    c = lax.axis_index("c")
    s = lax.axis_index("s")
    r0 = s * NODES_PER_TILE
    e_base = (c * N_TILES + s) * EDGES_PER_WORKER

    rows = pl.ds(r0, NODES_PER_TILE)
    w = c * N_TILES + s

    zeros16 = jnp.zeros((16,), jnp.float32)

    def zero_body(i, carry):
        deg_t[pl.ds(i * 16, 16)] = zeros16
        return carry

    lax.fori_loop(0, N_PAD // 16, zero_body, 0)

    @pl.when(c == 0)
    def _():
        pltpu.sync_copy(h.at[rows], acc_sh.at[rows])

    @pl.when(c == 1)
    def _():
        pltpu.sync_copy(zeros_h.at[rows], acc_sh.at[rows])

    plsc.subcore_barrier()

    SRC = [s0, s1, s2, s3]
    DST = [d0, d1, d2, d3]
    MSGS = [m0, m1]
    SEMI = [i0, i1, i2, i3]
    SEMG = [g0, g1]
    ones16 = jnp.ones((16,), jnp.float32)
    CPW = CHUNKS_PER_WORKER

    def fire_idx(j):
        q = j % 4
        e = e_base + j * CHUNK
        return (pltpu.async_copy(src1d.at[pl.ds(e, CHUNK)], SRC[q], SEMI[q]),
                pltpu.async_copy(dst1d.at[pl.ds(e, CHUNK)], DST[q], SEMI[q]))

    def fire_gather(j):
        return pltpu.async_copy(h.at[SRC[j % 4]], MSGS[j % 2], SEMG[j % 2])

    idx_d = {0: fire_idx(0), 1: fire_idx(1)}
    for dd in idx_d[0]:
        dd.wait()
    g_d = {0: fire_gather(0)}
    for j in range(CPW):
        g_d[j].wait()
        if j + 2 < CPW:
            idx_d[j + 2] = fire_idx(j + 2)
        if j + 1 < CPW:
            for dd in idx_d[j + 1]:
                dd.wait()
            g_d[j + 1] = fire_gather(j + 1)
        pltpu.sync_copy(MSGS[j % 2], acc_sh.at[DST[j % 4]], add=True)
        dq = DST[j % 4]
        for k in range(CHUNK // 16):
            plsc.addupdate_scatter(deg_t, [dq[pl.ds(k * 16, 16)]], ones16)
    plsc.subcore_barrier()

    @pl.when(c == 0)
    def _():
        pltpu.sync_copy(acc_sh.at[rows], acc0_out.at[rows])

    @pl.when(c == 1)
    def _():
        pltpu.sync_copy(acc_sh.at[rows], acc1_out.at[rows])

    pltpu.sync_copy(deg_t, deg_out.at[w])


CB_BLOCK = 1024


def _combine_body(a0_ref, a1_ref, d_ref, o_ref):
    deg = jnp.sum(d_ref[...], axis=0)[:, None]
    scale = 1.0 / (deg + 1.0)
    o_ref[...] = jnp.maximum((a0_ref[...] + a1_ref[...]) * scale, 0.0)


_combine = pl.pallas_call(
    _combine_body,
    grid=(N_PAD // CB_BLOCK,),
    in_specs=[
        pl.BlockSpec((CB_BLOCK, D_OUT), lambda i: (i, 0)),
        pl.BlockSpec((CB_BLOCK, D_OUT), lambda i: (i, 0)),
        pl.BlockSpec((N_WORKERS, CB_BLOCK), lambda i: (0, i)),
    ],
    out_specs=pl.BlockSpec((CB_BLOCK, D_OUT), lambda i: (i, 0)),
    out_shape=jax.ShapeDtypeStruct((N_PAD, D_OUT), jnp.float32),
)


def kernel(feat, edge_index, W):
    feat_p = jnp.concatenate(
        [feat, jnp.zeros((N_PAD - N_NODES, D_IN), jnp.float32)])
    h = _matmul(feat_p, W)

    pad = E_PAD - N_EDGES
    pad_src = jnp.arange(pad, dtype=jnp.int32) % N_NODES
    src = jnp.concatenate([edge_index[0], pad_src])
    pad_dst = N_NODES + (jnp.arange(pad, dtype=jnp.int32) % (N_PAD - N_NODES))
    dst = jnp.concatenate([edge_index[1], pad_dst])
    zeros_h = jnp.zeros((N_PAD, D_OUT), jnp.float32)

    acc0, acc1, deg = _sc_scatter(h, src, dst, zeros_h)
    return _combine(acc0, acc1, deg)[:N_NODES]


# speedup vs baseline: 1.1315x; 1.1315x over previous
import functools

import jax
import jax.numpy as jnp
from jax import lax
from jax.experimental import pallas as pl
from jax.experimental.pallas import tpu as pltpu
from jax.experimental.pallas import tpu_sc as plsc

N_NODES = 10000
N_EDGES = 320000
D_IN = 128
D_OUT = 128

N_TILES = 16
N_WORKERS = 32
CHUNK = 128
CHUNKS_PER_WORKER = 80
E_PAD = N_WORKERS * CHUNKS_PER_WORKER * CHUNK
EDGES_PER_WORKER = E_PAD // N_WORKERS
N_PAD = 10240
NODES_PER_TILE = N_PAD // N_TILES

MM_BLOCK = 640


def _mm_body(f_ref, w_ref, o_ref):
    o_ref[...] = jnp.dot(f_ref[...], w_ref[...],
                         preferred_element_type=jnp.float32)


_matmul = pl.pallas_call(
    _mm_body,
    grid=(N_PAD // MM_BLOCK,),
    in_specs=[
        pl.BlockSpec((MM_BLOCK, D_IN), lambda i: (i, 0)),
        pl.BlockSpec((D_IN, D_OUT), lambda i: (0, 0)),
    ],
    out_specs=pl.BlockSpec((MM_BLOCK, D_OUT), lambda i: (i, 0)),
    out_shape=jax.ShapeDtypeStruct((N_PAD, D_OUT), jnp.float32),
)


_sc_mesh = plsc.VectorSubcoreMesh(core_axis_name="c", subcore_axis_name="s")


@functools.partial(
    pl.kernel,
    out_type=(
        jax.ShapeDtypeStruct((N_PAD, D_OUT), jnp.float32),
        jax.ShapeDtypeStruct((N_PAD, D_OUT), jnp.float32),
        jax.ShapeDtypeStruct((N_WORKERS, N_PAD), jnp.float32),
    ),
    mesh=_sc_mesh,
    compiler_params=pltpu.CompilerParams(needs_layout_passes=False),
    scratch_types=(
        [pltpu.VMEM((CHUNK,), jnp.int32)] * 4
        + [pltpu.VMEM((CHUNK,), jnp.int32)] * 4
        + [pltpu.VMEM((CHUNK, D_OUT), jnp.float32)] * 2
        + [
            pltpu.VMEM((N_PAD,), jnp.float32),
            pltpu.VMEM_SHARED((N_PAD, D_OUT), jnp.float32),
        ]
        + [pltpu.SemaphoreType.DMA] * 8
    ),
)
def _sc_scatter(h, src1d, dst1d, zeros_h,
                acc0_out, acc1_out, deg_out,
                s0, s1, s2, s3, d0, d1, d2, d3, m0, m1,
                deg_t, acc_sh,
                i0, i1, i2, i3, g0, g1, t0, t1):
    c = lax.axis_index("c")
    s = lax.axis_index("s")
    r0 = s * NODES_PER_TILE
    e_base = (c * N_TILES + s) * EDGES_PER_WORKER

    rows = pl.ds(r0, NODES_PER_TILE)
    w = c * N_TILES + s

    zeros16 = jnp.zeros((16,), jnp.float32)

    def zero_body(i, carry):
        deg_t[pl.ds(i * 16, 16)] = zeros16
        return carry

    lax.fori_loop(0, N_PAD // 16, zero_body, 0)

    @pl.when(c == 0)
    def _():
        pltpu.sync_copy(h.at[rows], acc_sh.at[rows])

    @pl.when(c == 1)
    def _():
        pltpu.sync_copy(zeros_h.at[rows], acc_sh.at[rows])

    plsc.subcore_barrier()

    SRC = [s0, s1, s2, s3]
    DST = [d0, d1, d2, d3]
    MSGS = [m0, m1]
    SEMI = [i0, i1, i2, i3]
    SEMG = [g0, g1]
    ones16 = jnp.ones((16,), jnp.float32)
    CPW = CHUNKS_PER_WORKER

    def fire_idx(j):
        q = j % 4
        e = e_base + j * CHUNK
        return (pltpu.async_copy(src1d.at[pl.ds(e, CHUNK)], SRC[q], SEMI[q]),
                pltpu.async_copy(dst1d.at[pl.ds(e, CHUNK)], DST[q], SEMI[q]))

    def fire_gather(j):
        return pltpu.async_copy(h.at[SRC[j % 4]], MSGS[j % 2], SEMG[j % 2])

    idx_d = {0: fire_idx(0), 1: fire_idx(1)}
    for dd in idx_d[0]:
        dd.wait()
    g_d = {0: fire_gather(0)}
    for j in range(CPW):
        if j + 2 < CPW:
            idx_d[j + 2] = fire_idx(j + 2)
        if j + 1 < CPW:
            for dd in idx_d[j + 1]:
                dd.wait()
            g_d[j + 1] = fire_gather(j + 1)
        g_d[j].wait()
        pltpu.sync_copy(MSGS[j % 2], acc_sh.at[DST[j % 4]], add=True)
        dq = DST[j % 4]
        for k in range(CHUNK // 16):
            plsc.addupdate_scatter(deg_t, [dq[pl.ds(k * 16, 16)]], ones16)
    plsc.subcore_barrier()

    @pl.when(c == 0)
    def _():
        pltpu.sync_copy(acc_sh.at[rows], acc0_out.at[rows])

    @pl.when(c == 1)
    def _():
        pltpu.sync_copy(acc_sh.at[rows], acc1_out.at[rows])

    pltpu.sync_copy(deg_t, deg_out.at[w])


CB_BLOCK = 1024


def _combine_body(a0_ref, a1_ref, d_ref, o_ref):
    deg = jnp.sum(d_ref[...], axis=0)[:, None]
    scale = 1.0 / (deg + 1.0)
    o_ref[...] = jnp.maximum((a0_ref[...] + a1_ref[...]) * scale, 0.0)


_combine = pl.pallas_call(
    _combine_body,
    grid=(N_PAD // CB_BLOCK,),
    in_specs=[
        pl.BlockSpec((CB_BLOCK, D_OUT), lambda i: (i, 0)),
        pl.BlockSpec((CB_BLOCK, D_OUT), lambda i: (i, 0)),
        pl.BlockSpec((N_WORKERS, CB_BLOCK), lambda i: (0, i)),
    ],
    out_specs=pl.BlockSpec((CB_BLOCK, D_OUT), lambda i: (i, 0)),
    out_shape=jax.ShapeDtypeStruct((N_PAD, D_OUT), jnp.float32),
)


def kernel(feat, edge_index, W):
    feat_p = jnp.concatenate(
        [feat, jnp.zeros((N_PAD - N_NODES, D_IN), jnp.float32)])
    h = _matmul(feat_p, W)

    pad = E_PAD - N_EDGES
    pad_src = jnp.arange(pad, dtype=jnp.int32) % N_NODES
    src = jnp.concatenate([edge_index[0], pad_src])
    pad_dst = N_NODES + (jnp.arange(pad, dtype=jnp.int32) % (N_PAD - N_NODES))
    dst = jnp.concatenate([edge_index[1], pad_dst])
    zeros_h = jnp.zeros((N_PAD, D_OUT), jnp.float32)

    acc0, acc1, deg = _sc_scatter(h, src, dst, zeros_h)
    return _combine(acc0, acc1, deg)[:N_NODES]


# speedup vs baseline: 1.2532x; 1.1076x over previous
import functools

import jax
import jax.numpy as jnp
from jax import lax
from jax.experimental import pallas as pl
from jax.experimental.pallas import tpu as pltpu
from jax.experimental.pallas import tpu_sc as plsc

N_NODES = 10000
N_EDGES = 320000
D_IN = 128
D_OUT = 128

N_TILES = 16
N_WORKERS = 32
CHUNK = 128
CHUNKS_PER_WORKER = 80
E_PAD = N_WORKERS * CHUNKS_PER_WORKER * CHUNK
EDGES_PER_WORKER = E_PAD // N_WORKERS
N_PAD = 10240
NODES_PER_TILE = N_PAD // N_TILES

_sc_mesh = plsc.VectorSubcoreMesh(core_axis_name="c", subcore_axis_name="s")


@functools.partial(
    pl.kernel,
    out_type=(
        jax.ShapeDtypeStruct((N_PAD, D_IN), jnp.float32),
        jax.ShapeDtypeStruct((N_PAD, D_IN), jnp.float32),
        jax.ShapeDtypeStruct((N_WORKERS, N_PAD), jnp.float32),
    ),
    mesh=_sc_mesh,
    compiler_params=pltpu.CompilerParams(needs_layout_passes=False),
    scratch_types=(
        [pltpu.VMEM((CHUNK,), jnp.int32)] * 4
        + [pltpu.VMEM((CHUNK,), jnp.int32)] * 4
        + [pltpu.VMEM((CHUNK, D_IN), jnp.float32)] * 2
        + [
            pltpu.VMEM((N_PAD,), jnp.float32),
            pltpu.VMEM_SHARED((N_PAD, D_IN), jnp.float32),
        ]
        + [pltpu.SemaphoreType.DMA] * 6
    ),
)
def _sc_scatter(feat, src1d, dst1d,
                acc0_out, acc1_out, deg_out,
                s0, s1, s2, s3, d0, d1, d2, d3, m0, m1,
                deg_t, acc_sh,
                i0, i1, i2, i3, g0, g1):
    c = lax.axis_index("c")
    s = lax.axis_index("s")
    r0 = s * NODES_PER_TILE
    e_base = (c * N_TILES + s) * EDGES_PER_WORKER

    rows = pl.ds(r0, NODES_PER_TILE)
    w = c * N_TILES + s

    zeros16 = jnp.zeros((16,), jnp.float32)
    ones16 = jnp.ones((16,), jnp.float32)

    def zero_deg(i, carry):
        deg_t[pl.ds(i * 16, 16)] = zeros16
        return carry

    lax.fori_loop(0, N_PAD // 16, zero_deg, 0)

    def zero_m0(r, carry):
        for k in range(D_IN // 16):
            m0[r, pl.ds(k * 16, 16)] = zeros16
        return carry

    lax.fori_loop(0, CHUNK, zero_m0, 0)

    @pl.when(jnp.logical_and(c == 0, s < N_TILES - 1))
    def _():
        pltpu.sync_copy(feat.at[rows], acc_sh.at[rows])

    @pl.when(jnp.logical_and(c == 0, s == N_TILES - 1))
    def _():
        tail = N_NODES - (N_TILES - 1) * NODES_PER_TILE
        pltpu.sync_copy(feat.at[pl.ds(r0, tail)], acc_sh.at[pl.ds(r0, tail)])
        pltpu.sync_copy(m0, acc_sh.at[pl.ds(N_NODES, CHUNK)])
        pltpu.sync_copy(m0.at[pl.ds(0, N_PAD - N_NODES - CHUNK)],
                        acc_sh.at[pl.ds(N_NODES + CHUNK,
                                        N_PAD - N_NODES - CHUNK)])

    @pl.when(c == 1)
    def _():
        for t in range(NODES_PER_TILE // CHUNK):
            pltpu.sync_copy(m0, acc_sh.at[pl.ds(r0 + t * CHUNK, CHUNK)])

    plsc.subcore_barrier()

    SRC = [s0, s1, s2, s3]
    DST = [d0, d1, d2, d3]
    MSGS = [m0, m1]
    SEMI = [i0, i1, i2, i3]
    SEMG = [g0, g1]
    CPW = CHUNKS_PER_WORKER

    def fire_idx(j):
        q = j % 4
        e = e_base + j * CHUNK
        return (pltpu.async_copy(src1d.at[pl.ds(e, CHUNK)], SRC[q], SEMI[q]),
                pltpu.async_copy(dst1d.at[pl.ds(e, CHUNK)], DST[q], SEMI[q]))

    def fire_gather(j):
        return pltpu.async_copy(feat.at[SRC[j % 4]], MSGS[j % 2],
                                SEMG[j % 2])

    idx_d = {0: fire_idx(0), 1: fire_idx(1)}
    for dd in idx_d[0]:
        dd.wait()
    g_d = {0: fire_gather(0)}
    for j in range(CPW):
        if j + 2 < CPW:
            idx_d[j + 2] = fire_idx(j + 2)
        if j + 1 < CPW:
            for dd in idx_d[j + 1]:
                dd.wait()
            g_d[j + 1] = fire_gather(j + 1)
        g_d[j].wait()
        pltpu.sync_copy(MSGS[j % 2], acc_sh.at[DST[j % 4]], add=True)
        dq = DST[j % 4]
        for k in range(CHUNK // 16):
            plsc.addupdate_scatter(deg_t, [dq[pl.ds(k * 16, 16)]], ones16)
    plsc.subcore_barrier()

    @pl.when(c == 0)
    def _():
        pltpu.sync_copy(acc_sh.at[rows], acc0_out.at[rows])

    @pl.when(c == 1)
    def _():
        pltpu.sync_copy(acc_sh.at[rows], acc1_out.at[rows])

    pltpu.sync_copy(deg_t, deg_out.at[w])


CB_BLOCK = 1024


def _finish_body(a0_ref, a1_ref, d_ref, w_ref, o_ref):
    agg = a0_ref[...] + a1_ref[...]
    deg = jnp.sum(d_ref[...], axis=0)[:, None]
    h = jnp.dot(agg, w_ref[...], preferred_element_type=jnp.float32)
    o_ref[...] = jnp.maximum(h / (deg + 1.0), 0.0)


_finish = pl.pallas_call(
    _finish_body,
    grid=(N_PAD // CB_BLOCK,),
    in_specs=[
        pl.BlockSpec((CB_BLOCK, D_IN), lambda i: (i, 0)),
        pl.BlockSpec((CB_BLOCK, D_IN), lambda i: (i, 0)),
        pl.BlockSpec((N_WORKERS, CB_BLOCK), lambda i: (0, i)),
        pl.BlockSpec((D_IN, D_OUT), lambda i: (0, 0)),
    ],
    out_specs=pl.BlockSpec((CB_BLOCK, D_OUT), lambda i: (i, 0)),
    out_shape=jax.ShapeDtypeStruct((N_PAD, D_OUT), jnp.float32),
)


def kernel(feat, edge_index, W):
    pad = E_PAD - N_EDGES
    pad_src = jnp.arange(pad, dtype=jnp.int32) % N_NODES
    pad_dst = N_NODES + (jnp.arange(pad, dtype=jnp.int32) % (N_PAD - N_NODES))
    src = jnp.concatenate([edge_index[0], pad_src])
    dst = jnp.concatenate([edge_index[1], pad_dst])

    acc0, acc1, deg = _sc_scatter(feat, src, dst)
    return _finish(acc0, acc1, deg, W)[:N_NODES]


# speedup vs baseline: 1.2915x; 1.0306x over previous
import functools

import jax
import jax.numpy as jnp
from jax import lax
from jax.experimental import pallas as pl
from jax.experimental.pallas import tpu as pltpu
from jax.experimental.pallas import tpu_sc as plsc

N_NODES = 10000
N_EDGES = 320000
D_IN = 128
D_OUT = 128

N_TILES = 16
N_WORKERS = 32
CHUNK = 128
CHUNKS_PER_WORKER = 80
E_PAD = N_WORKERS * CHUNKS_PER_WORKER * CHUNK
EDGES_PER_WORKER = E_PAD // N_WORKERS
N_PAD = 10240
NODES_PER_TILE = N_PAD // N_TILES

_sc_mesh = plsc.VectorSubcoreMesh(core_axis_name="c", subcore_axis_name="s")


@functools.partial(
    pl.kernel,
    out_type=(
        jax.ShapeDtypeStruct((N_PAD, D_IN), jnp.float32),
        jax.ShapeDtypeStruct((N_PAD, D_IN), jnp.float32),
        jax.ShapeDtypeStruct((N_WORKERS, N_PAD), jnp.float32),
    ),
    mesh=_sc_mesh,
    compiler_params=pltpu.CompilerParams(needs_layout_passes=False),
    scratch_types=(
        [pltpu.VMEM((CHUNK,), jnp.int32)] * 4
        + [pltpu.VMEM((CHUNK,), jnp.int32)] * 4
        + [pltpu.VMEM((CHUNK, D_IN), jnp.float32)] * 2
        + [
            pltpu.VMEM((N_PAD,), jnp.float32),
            pltpu.VMEM_SHARED((N_PAD, D_IN), jnp.float32),
        ]
        + [pltpu.SemaphoreType.DMA] * 8
    ),
)
def _sc_scatter(feat, src1d, dst1d,
                acc0_out, acc1_out, deg_out,
                s0, s1, s2, s3, d0, d1, d2, d3, m0, m1,
                deg_t, acc_sh,
                i0, i1, i2, i3, g0, g1, t0, t1):
    c = lax.axis_index("c")
    s = lax.axis_index("s")
    r0 = s * NODES_PER_TILE
    e_base = (c * N_TILES + s) * EDGES_PER_WORKER

    rows = pl.ds(r0, NODES_PER_TILE)
    w = c * N_TILES + s

    zeros16 = jnp.zeros((16,), jnp.float32)
    ones16 = jnp.ones((16,), jnp.float32)

    def zero_deg(i, carry):
        deg_t[pl.ds(i * 16, 16)] = zeros16
        return carry

    lax.fori_loop(0, N_PAD // 16, zero_deg, 0)

    def zero_m0(r, carry):
        for k in range(D_IN // 16):
            m0[r, pl.ds(k * 16, 16)] = zeros16
        return carry

    lax.fori_loop(0, CHUNK, zero_m0, 0)

    @pl.when(jnp.logical_and(c == 0, s < N_TILES - 1))
    def _():
        pltpu.sync_copy(feat.at[rows], acc_sh.at[rows])

    @pl.when(jnp.logical_and(c == 0, s == N_TILES - 1))
    def _():
        tail = N_NODES - (N_TILES - 1) * NODES_PER_TILE
        pltpu.sync_copy(feat.at[pl.ds(r0, tail)], acc_sh.at[pl.ds(r0, tail)])
        pltpu.sync_copy(m0, acc_sh.at[pl.ds(N_NODES, CHUNK)])
        pltpu.sync_copy(m0.at[pl.ds(0, N_PAD - N_NODES - CHUNK)],
                        acc_sh.at[pl.ds(N_NODES + CHUNK,
                                        N_PAD - N_NODES - CHUNK)])

    @pl.when(c == 1)
    def _():
        for t in range(NODES_PER_TILE // CHUNK):
            pltpu.sync_copy(m0, acc_sh.at[pl.ds(r0 + t * CHUNK, CHUNK)])

    plsc.subcore_barrier()

    SRC = [s0, s1, s2, s3]
    DST = [d0, d1, d2, d3]
    MSGS = [m0, m1]
    SEMI = [i0, i1, i2, i3]
    SEMG = [g0, g1]
    SEMS = [t0, t1]
    CPW = CHUNKS_PER_WORKER

    def fire_idx(j):
        q = j % 4
        e = e_base + j * CHUNK
        return (pltpu.async_copy(src1d.at[pl.ds(e, CHUNK)], SRC[q], SEMI[q]),
                pltpu.async_copy(dst1d.at[pl.ds(e, CHUNK)], DST[q], SEMI[q]))

    def fire_gather(j):
        return pltpu.async_copy(feat.at[SRC[j % 4]], MSGS[j % 2],
                                SEMG[j % 2])

    def fire_scatter(j):
        return pltpu.async_copy(MSGS[j % 2], acc_sh.at[DST[j % 4]],
                                SEMS[j % 2], add=True)

    idx_d = {0: fire_idx(0), 1: fire_idx(1)}
    for dd in idx_d[0]:
        dd.wait()
    g_d = {0: fire_gather(0)}
    s_d = {}
    for j in range(CPW):
        if j + 2 < CPW:
            idx_d[j + 2] = fire_idx(j + 2)
        if j + 1 < CPW:
            for dd in idx_d[j + 1]:
                dd.wait()
        if j >= 1:
            s_d[j - 1].wait()
        if j + 1 < CPW:
            g_d[j + 1] = fire_gather(j + 1)
        g_d[j].wait()
        s_d[j] = fire_scatter(j)
        dq = DST[j % 4]
        for k in range(CHUNK // 16):
            plsc.addupdate_scatter(deg_t, [dq[pl.ds(k * 16, 16)]], ones16)
    s_d[CPW - 1].wait()
    plsc.subcore_barrier()

    @pl.when(c == 0)
    def _():
        pltpu.sync_copy(acc_sh.at[rows], acc0_out.at[rows])

    @pl.when(c == 1)
    def _():
        pltpu.sync_copy(acc_sh.at[rows], acc1_out.at[rows])

    pltpu.sync_copy(deg_t, deg_out.at[w])


CB_BLOCK = 1024


def _finish_body(a0_ref, a1_ref, d_ref, w_ref, o_ref):
    agg = a0_ref[...] + a1_ref[...]
    deg = jnp.sum(d_ref[...], axis=0)[:, None]
    h = jnp.dot(agg, w_ref[...], preferred_element_type=jnp.float32)
    o_ref[...] = jnp.maximum(h / (deg + 1.0), 0.0)


_finish = pl.pallas_call(
    _finish_body,
    grid=(N_PAD // CB_BLOCK,),
    in_specs=[
        pl.BlockSpec((CB_BLOCK, D_IN), lambda i: (i, 0)),
        pl.BlockSpec((CB_BLOCK, D_IN), lambda i: (i, 0)),
        pl.BlockSpec((N_WORKERS, CB_BLOCK), lambda i: (0, i)),
        pl.BlockSpec((D_IN, D_OUT), lambda i: (0, 0)),
    ],
    out_specs=pl.BlockSpec((CB_BLOCK, D_OUT), lambda i: (i, 0)),
    out_shape=jax.ShapeDtypeStruct((N_NODES, D_OUT), jnp.float32),
)


def kernel(feat, edge_index, W):
    pad = E_PAD - N_EDGES
    pad_src = jnp.arange(pad, dtype=jnp.int32) % N_NODES
    pad_dst = N_NODES + (jnp.arange(pad, dtype=jnp.int32) % (N_PAD - N_NODES))
    src = jnp.concatenate([edge_index[0], pad_src])
    dst = jnp.concatenate([edge_index[1], pad_dst])

    acc0, acc1, deg = _sc_scatter(feat, src, dst)
    return _finish(acc0, acc1, deg, W)


# speedup vs baseline: 1.2995x; 1.0061x over previous
import functools

import jax
import jax.numpy as jnp
from jax import lax
from jax.experimental import pallas as pl
from jax.experimental.pallas import tpu as pltpu
from jax.experimental.pallas import tpu_sc as plsc

N_NODES = 10000
N_EDGES = 320000
D_IN = 128
D_OUT = 128

N_TILES = 16
N_WORKERS = 32
CHUNK = 128
EDGES_PER_WORKER = N_EDGES // N_WORKERS
CHUNKS_PER_WORKER = EDGES_PER_WORKER // CHUNK
TAIL = EDGES_PER_WORKER - CHUNKS_PER_WORKER * CHUNK
N_PAD = 10240
NODES_PER_TILE = N_PAD // N_TILES

_sc_mesh = plsc.VectorSubcoreMesh(core_axis_name="c", subcore_axis_name="s")


@functools.partial(
    pl.kernel,
    out_type=(
        jax.ShapeDtypeStruct((N_PAD, D_IN), jnp.float32),
        jax.ShapeDtypeStruct((N_PAD, D_IN), jnp.float32),
        jax.ShapeDtypeStruct((N_WORKERS, N_PAD), jnp.float32),
    ),
    mesh=_sc_mesh,
    compiler_params=pltpu.CompilerParams(needs_layout_passes=False),
    scratch_types=(
        [pltpu.VMEM((CHUNK,), jnp.int32)] * 4
        + [pltpu.VMEM((CHUNK,), jnp.int32)] * 4
        + [pltpu.VMEM((CHUNK, D_IN), jnp.float32)] * 2
        + [
            pltpu.VMEM((TAIL,), jnp.int32),
            pltpu.VMEM((TAIL,), jnp.int32),
            pltpu.VMEM((TAIL, D_IN), jnp.float32),
            pltpu.VMEM((N_PAD,), jnp.float32),
            pltpu.VMEM_SHARED((N_PAD, D_IN), jnp.float32),
        ]
        + [pltpu.SemaphoreType.DMA] * 8
    ),
)
def _sc_scatter(feat, src1d, dst1d,
                acc0_out, acc1_out, deg_out,
                s0, s1, s2, s3, d0, d1, d2, d3, m0, m1,
                st, dt, mt, deg_t, acc_sh,
                i0, i1, i2, i3, g0, g1, t0, t1):
    c = lax.axis_index("c")
    s = lax.axis_index("s")
    r0 = s * NODES_PER_TILE
    e_base = (c * N_TILES + s) * EDGES_PER_WORKER

    rows = pl.ds(r0, NODES_PER_TILE)
    w = c * N_TILES + s

    zeros16 = jnp.zeros((16,), jnp.float32)
    ones16 = jnp.ones((16,), jnp.float32)

    def zero_deg(i, carry):
        deg_t[pl.ds(i * 16, 16)] = zeros16
        return carry

    lax.fori_loop(0, N_PAD // 16, zero_deg, 0)

    def zero_m0(r, carry):
        for k in range(D_IN // 16):
            m0[r, pl.ds(k * 16, 16)] = zeros16
        return carry

    lax.fori_loop(0, CHUNK, zero_m0, 0)

    @pl.when(jnp.logical_and(c == 0, s < N_TILES - 1))
    def _():
        pltpu.sync_copy(feat.at[rows], acc_sh.at[rows])

    @pl.when(jnp.logical_and(c == 0, s == N_TILES - 1))
    def _():
        tail = N_NODES - (N_TILES - 1) * NODES_PER_TILE
        pltpu.sync_copy(feat.at[pl.ds(r0, tail)], acc_sh.at[pl.ds(r0, tail)])
        pltpu.sync_copy(m0, acc_sh.at[pl.ds(N_NODES, CHUNK)])
        pltpu.sync_copy(m0.at[pl.ds(0, N_PAD - N_NODES - CHUNK)],
                        acc_sh.at[pl.ds(N_NODES + CHUNK,
                                        N_PAD - N_NODES - CHUNK)])

    @pl.when(c == 1)
    def _():
        for t in range(NODES_PER_TILE // CHUNK):
            pltpu.sync_copy(m0, acc_sh.at[pl.ds(r0 + t * CHUNK, CHUNK)])

    plsc.subcore_barrier()

    SRC = [s0, s1, s2, s3]
    DST = [d0, d1, d2, d3]
    MSGS = [m0, m1]
    SEMI = [i0, i1, i2, i3]
    SEMG = [g0, g1]
    SEMS = [t0, t1]
    CPW = CHUNKS_PER_WORKER

    def fire_idx(j):
        q = j % 4
        e = e_base + j * CHUNK
        return (pltpu.async_copy(src1d.at[pl.ds(e, CHUNK)], SRC[q], SEMI[q]),
                pltpu.async_copy(dst1d.at[pl.ds(e, CHUNK)], DST[q], SEMI[q]))

    def fire_gather(j):
        return pltpu.async_copy(feat.at[SRC[j % 4]], MSGS[j % 2],
                                SEMG[j % 2])

    def fire_scatter(j):
        return pltpu.async_copy(MSGS[j % 2], acc_sh.at[DST[j % 4]],
                                SEMS[j % 2], add=True)

    idx_d = {0: fire_idx(0), 1: fire_idx(1)}
    for dd in idx_d[0]:
        dd.wait()
    g_d = {0: fire_gather(0)}
    s_d = {}
    for j in range(CPW):
        if j + 2 < CPW:
            idx_d[j + 2] = fire_idx(j + 2)
        if j + 1 < CPW:
            for dd in idx_d[j + 1]:
                dd.wait()
        if j >= 1:
            s_d[j - 1].wait()
        if j + 1 < CPW:
            g_d[j + 1] = fire_gather(j + 1)
        g_d[j].wait()
        s_d[j] = fire_scatter(j)
        dq = DST[j % 4]
        for k in range(CHUNK // 16):
            plsc.addupdate_scatter(deg_t, [dq[pl.ds(k * 16, 16)]], ones16)
    s_d[CPW - 1].wait()

    e_t = e_base + CPW * CHUNK
    pltpu.sync_copy(src1d.at[pl.ds(e_t, TAIL)], st)
    pltpu.sync_copy(dst1d.at[pl.ds(e_t, TAIL)], dt)
    pltpu.async_copy(feat.at[st], mt, g0).wait()
    pltpu.sync_copy(mt, acc_sh.at[dt], add=True)
    plsc.addupdate_scatter(deg_t, [dt[...]], ones16)
    plsc.subcore_barrier()

    @pl.when(c == 0)
    def _():
        pltpu.sync_copy(acc_sh.at[rows], acc0_out.at[rows])

    @pl.when(c == 1)
    def _():
        pltpu.sync_copy(acc_sh.at[rows], acc1_out.at[rows])

    pltpu.sync_copy(deg_t, deg_out.at[w])


CB_BLOCK = 1024


def _finish_body(a0_ref, a1_ref, d_ref, w_ref, o_ref):
    agg = a0_ref[...] + a1_ref[...]
    deg = jnp.sum(d_ref[...], axis=0)[:, None]
    h = jnp.dot(agg, w_ref[...], preferred_element_type=jnp.float32)
    o_ref[...] = jnp.maximum(h / (deg + 1.0), 0.0)


_finish = pl.pallas_call(
    _finish_body,
    grid=(N_PAD // CB_BLOCK,),
    in_specs=[
        pl.BlockSpec((CB_BLOCK, D_IN), lambda i: (i, 0)),
        pl.BlockSpec((CB_BLOCK, D_IN), lambda i: (i, 0)),
        pl.BlockSpec((N_WORKERS, CB_BLOCK), lambda i: (0, i)),
        pl.BlockSpec((D_IN, D_OUT), lambda i: (0, 0)),
    ],
    out_specs=pl.BlockSpec((CB_BLOCK, D_OUT), lambda i: (i, 0)),
    out_shape=jax.ShapeDtypeStruct((N_NODES, D_OUT), jnp.float32),
)


def kernel(feat, edge_index, W):
    acc0, acc1, deg = _sc_scatter(feat, edge_index[0], edge_index[1])
    return _finish(acc0, acc1, deg, W)


# speedup vs baseline: 1.3737x; 1.0571x over previous
import functools

import jax
import jax.numpy as jnp
from jax import lax
from jax.experimental import pallas as pl
from jax.experimental.pallas import tpu as pltpu
from jax.experimental.pallas import tpu_sc as plsc

N_NODES = 10000
N_EDGES = 320000
D_IN = 128
D_OUT = 128

N_TILES = 16
N_WORKERS = 32
CHUNK = 96
EDGES_PER_WORKER = N_EDGES // N_WORKERS
CHUNKS_PER_WORKER = EDGES_PER_WORKER // CHUNK
TAIL = EDGES_PER_WORKER - CHUNKS_PER_WORKER * CHUNK
N_PAD = 10240
NODES_PER_TILE = N_PAD // N_TILES

_sc_mesh = plsc.VectorSubcoreMesh(core_axis_name="c", subcore_axis_name="s")


@functools.partial(
    pl.kernel,
    out_type=(
        jax.ShapeDtypeStruct((N_PAD, D_IN), jnp.float32),
        jax.ShapeDtypeStruct((N_PAD, D_IN), jnp.float32),
        jax.ShapeDtypeStruct((N_WORKERS, N_PAD), jnp.float32),
    ),
    mesh=_sc_mesh,
    compiler_params=pltpu.CompilerParams(needs_layout_passes=False),
    scratch_types=(
        [pltpu.VMEM((CHUNK,), jnp.int32)] * 6
        + [pltpu.VMEM((CHUNK,), jnp.int32)] * 6
        + [pltpu.VMEM((CHUNK, D_IN), jnp.float32)] * 3
        + [
            pltpu.VMEM((TAIL,), jnp.int32),
            pltpu.VMEM((TAIL,), jnp.int32),
            pltpu.VMEM((N_PAD,), jnp.float32),
            pltpu.VMEM_SHARED((N_PAD, D_IN), jnp.float32),
        ]
        + [pltpu.SemaphoreType.DMA] * 11
    ),
)
def _sc_scatter(feat, src1d, dst1d,
                acc0_out, acc1_out, deg_out,
                s0, s1, s2, s3, s4, s5, d0, d1, d2, d3, d4, d5, m0, m1, m2,
                st, dt, deg_t, acc_sh,
                i0, i1, i2, i3, i4, i5, g0, g1, g2, t0, t1):
    c = lax.axis_index("c")
    s = lax.axis_index("s")
    r0 = s * NODES_PER_TILE
    e_base = (c * N_TILES + s) * EDGES_PER_WORKER

    rows = pl.ds(r0, NODES_PER_TILE)
    w = c * N_TILES + s

    zeros16 = jnp.zeros((16,), jnp.float32)
    ones16 = jnp.ones((16,), jnp.float32)

    def zero_deg(i, carry):
        deg_t[pl.ds(i * 16, 16)] = zeros16
        return carry

    lax.fori_loop(0, N_PAD // 16, zero_deg, 0)

    def zero_m0(r, carry):
        for k in range(D_IN // 16):
            m0[r, pl.ds(k * 16, 16)] = zeros16
        return carry

    lax.fori_loop(0, CHUNK, zero_m0, 0)

    @pl.when(jnp.logical_and(c == 0, s < N_TILES - 1))
    def _():
        pltpu.sync_copy(feat.at[rows], acc_sh.at[rows])

    @pl.when(jnp.logical_and(c == 0, s == N_TILES - 1))
    def _():
        tail = N_NODES - (N_TILES - 1) * NODES_PER_TILE
        pltpu.sync_copy(feat.at[pl.ds(r0, tail)], acc_sh.at[pl.ds(r0, tail)])
        for t in range(3):
            pltpu.sync_copy(m0.at[pl.ds(0, 80)],
                            acc_sh.at[pl.ds(N_NODES + t * 80, 80)])

    @pl.when(c == 1)
    def _():
        for t in range(8):
            pltpu.sync_copy(m0.at[pl.ds(0, 80)],
                            acc_sh.at[pl.ds(r0 + t * 80, 80)])

    plsc.subcore_barrier()

    SRC = [s0, s1, s2, s3, s4, s5]
    DST = [d0, d1, d2, d3, d4, d5]
    MSGS = [m0, m1, m2]
    SEMI = [i0, i1, i2, i3, i4, i5]
    SEMG = [g0, g1, g2]
    SEMS = [t0, t1]
    CPW = CHUNKS_PER_WORKER

    def fire_idx(j):
        q = j % 6
        e = e_base + j * CHUNK
        return (pltpu.async_copy(src1d.at[pl.ds(e, CHUNK)], SRC[q], SEMI[q]),
                pltpu.async_copy(dst1d.at[pl.ds(e, CHUNK)], DST[q], SEMI[q]))

    def fire_gather(j):
        return pltpu.async_copy(feat.at[SRC[j % 6]], MSGS[j % 3],
                                SEMG[j % 3])

    def fire_scatter(j):
        return pltpu.async_copy(MSGS[j % 3], acc_sh.at[DST[j % 6]],
                                SEMS[j % 2], add=True)

    idx_d = {j: fire_idx(j) for j in range(4)}
    for j in (0, 1):
        for dd in idx_d[j]:
            dd.wait()
    g_d = {0: fire_gather(0), 1: fire_gather(1)}
    s_d = {}
    for j in range(CPW):
        if j >= 1:
            s_d[j - 1].wait()
        if j + 2 < CPW:
            for dd in idx_d[j + 2]:
                dd.wait()
            g_d[j + 2] = fire_gather(j + 2)
        g_d[j].wait()
        if j + 4 < CPW:
            idx_d[j + 4] = fire_idx(j + 4)
        s_d[j] = fire_scatter(j)
        dq = DST[j % 6]
        for k in range(CHUNK // 16):
            plsc.addupdate_scatter(deg_t, [dq[pl.ds(k * 16, 16)]], ones16)
    s_d[CPW - 1].wait()

    e_t = e_base + CPW * CHUNK
    pltpu.sync_copy(src1d.at[pl.ds(e_t, TAIL)], st)
    pltpu.sync_copy(dst1d.at[pl.ds(e_t, TAIL)], dt)
    mt = m0.at[pl.ds(0, TAIL)]
    pltpu.async_copy(feat.at[st], mt, g0).wait()
    pltpu.sync_copy(mt, acc_sh.at[dt], add=True)
    plsc.addupdate_scatter(deg_t, [dt[...]], ones16)
    plsc.subcore_barrier()

    @pl.when(c == 0)
    def _():
        pltpu.sync_copy(acc_sh.at[rows], acc0_out.at[rows])

    @pl.when(c == 1)
    def _():
        pltpu.sync_copy(acc_sh.at[rows], acc1_out.at[rows])

    pltpu.sync_copy(deg_t, deg_out.at[w])


CB_BLOCK = 1024


def _finish_body(a0_ref, a1_ref, d_ref, w_ref, o_ref):
    agg = a0_ref[...] + a1_ref[...]
    deg = jnp.sum(d_ref[...], axis=0)[:, None]
    h = jnp.dot(agg, w_ref[...], preferred_element_type=jnp.float32)
    o_ref[...] = jnp.maximum(h / (deg + 1.0), 0.0)


_finish = pl.pallas_call(
    _finish_body,
    grid=(N_PAD // CB_BLOCK,),
    in_specs=[
        pl.BlockSpec((CB_BLOCK, D_IN), lambda i: (i, 0)),
        pl.BlockSpec((CB_BLOCK, D_IN), lambda i: (i, 0)),
        pl.BlockSpec((N_WORKERS, CB_BLOCK), lambda i: (0, i)),
        pl.BlockSpec((D_IN, D_OUT), lambda i: (0, 0)),
    ],
    out_specs=pl.BlockSpec((CB_BLOCK, D_OUT), lambda i: (i, 0)),
    out_shape=jax.ShapeDtypeStruct((N_NODES, D_OUT), jnp.float32),
)


def kernel(feat, edge_index, W):
    acc0, acc1, deg = _sc_scatter(feat, edge_index[0], edge_index[1])
    return _finish(acc0, acc1, deg, W)
